# Initial kernel scaffold; baseline (speedup 1.0000x reference)
import jax
import jax.numpy as jnp
from jax.experimental import pallas as pl


def kernel(x, conn, w1, w2):
    raise NotImplementedError("write your pallas kernel here")


# speedup vs baseline: 7.6288x; 7.6288x over previous
import jax
import jax.numpy as jnp
from jax import lax
from jax.experimental import pallas as pl
from jax.experimental.pallas import tpu as pltpu, tpu_sc as plsc

B = 4
C = 96
OC = 96
HO = 112
WO = 112
HP = 113
WPAD = 128
R = 28
NCHUNK = HO // R
NWORK = 32
PAIRS_PER_W = (B * OC) // NWORK


def _sc_body(x_hbm, q_hbm, roff_hbm, coff_hbm, w1_hbm, w2_hbm, out_hbm,
             q_v, roff_v, coff_v, w1_v, w2_v, tap_v, out_v, sem_a, sem_b,
             osem):
    cid = lax.axis_index("c")
    sid = lax.axis_index("s")
    wid = sid * 2 + cid

    pltpu.sync_copy(q_hbm, q_v)
    pltpu.sync_copy(roff_hbm, roff_v)
    pltpu.sync_copy(coff_hbm, coff_v)
    pltpu.sync_copy(w1_hbm, w1_v)
    pltpu.sync_copy(w2_hbm, w2_v)

    def pair_body(k, _):
        b = k & 3
        oc = wid * 3 + lax.shift_right_logical(k, 2)
        base = oc * 9

        qvec = q_v[pl.ds(base, 16)]
        rvec = roff_v[pl.ds(base, 16)]
        cvec = coff_v[pl.ds(base, 16)]
        w1vec = w1_v[pl.ds(base, 16)]
        w2vec = w2_v[pl.ds(oc * 3, 16)]
        planes = [qvec[t] + b * C for t in range(9)]
        roffs = [rvec[t] for t in range(9)]
        coffs = [cvec[t] for t in range(9)]
        w1s = [jnp.full((16,), w1vec[t], jnp.float32) for t in range(9)]
        w2s = [jnp.full((16,), w2vec[i], jnp.float32) for i in range(3)]

        def fetch(ch, slot, sem):
            h0 = ch * R
            return [pltpu.async_copy(
                x_hbm.at[planes[t], pl.ds(h0 + roffs[t], R), :],
                tap_v.at[slot, t], sem) for t in range(9)]

        def compute(slot, ch):
            def row_body(r, _):
                for cc in range(WO // 16):
                    acc = None
                    for i in range(3):
                        m = None
                        for j in range(3):
                            t = 3 * i + j
                            v = tap_v[slot, t, r, pl.ds(coffs[t] + cc * 16, 16)]
                            v = v - w1s[t]
                            m = v if m is None else jnp.maximum(m, v)
                        m = m - w2s[i]
                        acc = m if acc is None else jnp.minimum(acc, m)
                    out_v[slot, r, pl.ds(cc * 16, 16)] = acc
                return 0

            lax.fori_loop(0, R, row_body, 0)
            pltpu.async_copy(
                out_v.at[slot], out_hbm.at[b, oc, pl.ds(ch * R, R), :],
                osem).wait()

        cps = fetch(0, 0, sem_a)
        for ch2 in range(NCHUNK // 2):
            e = 2 * ch2
            cps_o = fetch(e + 1, 1, sem_b)
            for cp in cps:
                cp.wait()
            compute(0, e)
            cps = fetch(e + 2, 0, sem_a) if e + 2 < NCHUNK else []
            for cp in cps_o:
                cp.wait()
            compute(1, e + 1)
        return 0

    lax.fori_loop(0, PAIRS_PER_W, pair_body, 0)


def kernel(x, conn, w1, w2):
    xp = jnp.pad(x, ((0, 0), (0, 0), (1, 1), (1, 1)), mode="edge")
    X = xp.reshape(B, C, HP, 2, HP, 2).transpose(3, 5, 0, 1, 2, 4)
    X = jnp.pad(X, ((0, 0),) * 5 + ((0, WPAD - HP),))
    X = X.reshape(2 * 2 * B * C, HP, WPAD)

    conn = conn.astype(jnp.int32)
    c = conn // 9
    rem = conn - c * 9
    di = rem // 3
    dj = rem - di * 3
    qbase = ((di % 2) * 2 + dj % 2) * (B * C) + c
    pad16 = lambda a: jnp.pad(a, (0, 16))
    q_p = pad16(qbase)
    roff_p = pad16(di // 2)
    coff_p = pad16(dj // 2)
    w1_p = pad16(w1.reshape(-1))
    w2_p = pad16(w2.reshape(-1))

    mesh = plsc.VectorSubcoreMesh(core_axis_name="c", subcore_axis_name="s")
    run = pl.kernel(
        _sc_body,
        out_type=jax.ShapeDtypeStruct((B, OC, HO, WO), jnp.float32),
        mesh=mesh,
        compiler_params=pltpu.CompilerParams(use_tc_tiling_on_sc=False),
        scratch_types=[
            pltpu.VMEM((OC * 9 + 16,), jnp.int32),
            pltpu.VMEM((OC * 9 + 16,), jnp.int32),
            pltpu.VMEM((OC * 9 + 16,), jnp.int32),
            pltpu.VMEM((OC * 9 + 16,), jnp.float32),
            pltpu.VMEM((OC * 3 + 16,), jnp.float32),
            pltpu.VMEM((2, 9, R, WPAD), jnp.float32),
            pltpu.VMEM((2, R, WO), jnp.float32),
            pltpu.SemaphoreType.DMA,
            pltpu.SemaphoreType.DMA,
            pltpu.SemaphoreType.DMA,
        ],
    )
    return run(X, q_p, roff_p, coff_p, w1_p, w2_p)


# speedup vs baseline: 12.6888x; 1.6633x over previous
import jax
import jax.numpy as jnp
from jax import lax
from jax.experimental import pallas as pl
from jax.experimental.pallas import tpu as pltpu, tpu_sc as plsc

B = 4
C = 96
OC = 96
HO = 112
WO = 112
HP = 113
WPAD = 128
R = 28
NCHUNK = HO // R
NWORK = 32
PAIRS_PER_W = (B * OC) // NWORK


def _deint_body(x_hbm, xp_hbm, src_v, po_v, sem, osem):
    cid = lax.axis_index("c")
    sid = lax.axis_index("s")
    wid = sid * 2 + cid

    iota = lax.iota(jnp.int32, 16)
    i2 = iota + iota
    zero = jnp.zeros((16,), jnp.int32)
    c223 = jnp.full((16,), 223, jnp.int32)

    def pair_body(k, _):
        b = k & 3
        c = wid * 3 + lax.shift_right_logical(k, 2)
        pltpu.sync_copy(x_hbm.at[b, c], src_v)
        qb = b * C + c
        for pp in range(4):
            pr, pc = pp >> 1, pp & 1

            def row_body(i, _):
                s = jnp.minimum(jnp.maximum(i + i + (pr - 1), 0), 223)
                srow = jnp.full((16,), s, jnp.int32)
                for kv in range(WPAD // 16):
                    cidx = i2 + jnp.full((16,), 32 * kv + pc - 1, jnp.int32)
                    cidx = jnp.minimum(jnp.maximum(cidx, zero), c223)
                    po_v[pp & 1, i, pl.ds(kv * 16, 16)] = plsc.load_gather(
                        src_v, [srow, cidx])
                return 0

            lax.fori_loop(0, HP, row_body, 0)
            pltpu.async_copy(
                po_v.at[pp & 1], xp_hbm.at[pp * (B * C) + qb], osem).wait()
        return 0

    lax.fori_loop(0, PAIRS_PER_W, pair_body, 0)


def _sc_body(x_hbm, q_hbm, roff_hbm, coff_hbm, w1_hbm, w2_hbm, out_hbm,
             q_v, roff_v, coff_v, w1_v, w2_v, tap_v, out_v, sem_a, sem_b,
             osem):
    cid = lax.axis_index("c")
    sid = lax.axis_index("s")
    wid = sid * 2 + cid

    pltpu.sync_copy(q_hbm, q_v)
    pltpu.sync_copy(roff_hbm, roff_v)
    pltpu.sync_copy(coff_hbm, coff_v)
    pltpu.sync_copy(w1_hbm, w1_v)
    pltpu.sync_copy(w2_hbm, w2_v)

    def pair_body(k, _):
        b = k & 3
        oc = wid * 3 + lax.shift_right_logical(k, 2)
        base = oc * 9

        qvec = q_v[pl.ds(base, 16)]
        rvec = roff_v[pl.ds(base, 16)]
        cvec = coff_v[pl.ds(base, 16)]
        w1vec = w1_v[pl.ds(base, 16)]
        w2vec = w2_v[pl.ds(oc * 3, 16)]
        planes = [qvec[t] + b * C for t in range(9)]
        roffs = [rvec[t] for t in range(9)]
        coffs = [cvec[t] for t in range(9)]
        w1s = [jnp.full((16,), w1vec[t], jnp.float32) for t in range(9)]
        w2s = [jnp.full((16,), w2vec[i], jnp.float32) for i in range(3)]

        def fetch(ch, slot, sem):
            h0 = ch * R
            return [pltpu.async_copy(
                x_hbm.at[planes[t], pl.ds(h0 + roffs[t], R), :],
                tap_v.at[slot, t], sem) for t in range(9)]

        def compute(slot, ch):
            def row_body(r, _):
                for cc in range(WO // 16):
                    acc = None
                    for i in range(3):
                        m = None
                        for j in range(3):
                            t = 3 * i + j
                            v = tap_v[slot, t, r, pl.ds(coffs[t] + cc * 16, 16)]
                            v = v - w1s[t]
                            m = v if m is None else jnp.maximum(m, v)
                        m = m - w2s[i]
                        acc = m if acc is None else jnp.minimum(acc, m)
                    out_v[slot, r, pl.ds(cc * 16, 16)] = acc
                return 0

            lax.fori_loop(0, R, row_body, 0)
            pltpu.async_copy(
                out_v.at[slot], out_hbm.at[b, oc, pl.ds(ch * R, R), :],
                osem).wait()

        cps = fetch(0, 0, sem_a)
        for ch2 in range(NCHUNK // 2):
            e = 2 * ch2
            cps_o = fetch(e + 1, 1, sem_b)
            for cp in cps:
                cp.wait()
            compute(0, e)
            cps = fetch(e + 2, 0, sem_a) if e + 2 < NCHUNK else []
            for cp in cps_o:
                cp.wait()
            compute(1, e + 1)
        return 0

    lax.fori_loop(0, PAIRS_PER_W, pair_body, 0)


def kernel(x, conn, w1, w2):
    mesh = plsc.VectorSubcoreMesh(core_axis_name="c", subcore_axis_name="s")

    deint = pl.kernel(
        _deint_body,
        out_type=jax.ShapeDtypeStruct((2 * 2 * B * C, HP, WPAD), jnp.float32),
        mesh=mesh,
        compiler_params=pltpu.CompilerParams(use_tc_tiling_on_sc=False, needs_layout_passes=False),
        scratch_types=[
            pltpu.VMEM((224, 224), jnp.float32),
            pltpu.VMEM((2, HP, WPAD), jnp.float32),
            pltpu.SemaphoreType.DMA,
            pltpu.SemaphoreType.DMA,
        ],
    )
    X = deint(x)

    conn = conn.astype(jnp.int32)
    c = conn // 9
    rem = conn - c * 9
    di = rem // 3
    dj = rem - di * 3
    qbase = ((di % 2) * 2 + dj % 2) * (B * C) + c
    pad16 = lambda a: jnp.pad(a, (0, 16))
    q_p = pad16(qbase)
    roff_p = pad16(di // 2)
    coff_p = pad16(dj // 2)
    w1_p = pad16(w1.reshape(-1))
    w2_p = pad16(w2.reshape(-1))

    run = pl.kernel(
        _sc_body,
        out_type=jax.ShapeDtypeStruct((B, OC, HO, WO), jnp.float32),
        mesh=mesh,
        compiler_params=pltpu.CompilerParams(use_tc_tiling_on_sc=False, needs_layout_passes=False),
        scratch_types=[
            pltpu.VMEM((OC * 9 + 16,), jnp.int32),
            pltpu.VMEM((OC * 9 + 16,), jnp.int32),
            pltpu.VMEM((OC * 9 + 16,), jnp.int32),
            pltpu.VMEM((OC * 9 + 16,), jnp.float32),
            pltpu.VMEM((OC * 3 + 16,), jnp.float32),
            pltpu.VMEM((2, 9, R, WPAD), jnp.float32),
            pltpu.VMEM((2, R, WO), jnp.float32),
            pltpu.SemaphoreType.DMA,
            pltpu.SemaphoreType.DMA,
            pltpu.SemaphoreType.DMA,
        ],
    )
    return run(X, q_p, roff_p, coff_p, w1_p, w2_p)


# speedup vs baseline: 13.0063x; 1.0250x over previous
import jax
import jax.numpy as jnp
from jax import lax
from jax.experimental import pallas as pl
from jax.experimental.pallas import tpu as pltpu, tpu_sc as plsc

B = 4
C = 96
OC = 96
HO = 112
WO = 112
HP = 113
WPAD = 128
R = 28
NCHUNK = HO // R
NWORK = 32
PAIRS_PER_W = (B * OC) // NWORK


def _deint_body(x_hbm, xp_hbm, src_v, po_v, sem, osem):
    cid = lax.axis_index("c")
    sid = lax.axis_index("s")
    wid = sid * 2 + cid

    iota = lax.iota(jnp.int32, 16)
    i2 = iota + iota
    zero = jnp.zeros((16,), jnp.int32)
    c223 = jnp.full((16,), 223, jnp.int32)
    cidxs = [[jnp.minimum(jnp.maximum(
        i2 + jnp.full((16,), 32 * kv + pc - 1, jnp.int32), zero), c223)
        for kv in range(WPAD // 16)] for pc in range(2)]

    def drain_outs(qb):
        for pp in range(4):
            pltpu.make_async_copy(
                po_v.at[pp], xp_hbm.at[pp * (B * C) + qb], osem).wait()

    def pair_body(k, _):
        b = k & 3
        c = wid * 3 + lax.shift_right_logical(k, 2)
        pltpu.sync_copy(x_hbm.at[b, c], src_v)

        @pl.when(k > 0)
        def _():
            drain_outs(0)

        def row_body(i, _):
            s0 = jnp.maximum(i + i - 1, 0)
            s1 = jnp.minimum(i + i, 223)
            srows = (jnp.full((16,), s0, jnp.int32),
                     jnp.full((16,), s1, jnp.int32))
            for pp in range(4):
                pr, pc = pp >> 1, pp & 1
                for kv in range(WPAD // 16):
                    po_v[pp, i, pl.ds(kv * 16, 16)] = plsc.load_gather(
                        src_v, [srows[pr], cidxs[pc][kv]])
            return 0

        lax.fori_loop(0, HP, row_body, 0)
        qb = b * C + c
        for pp in range(4):
            pltpu.async_copy(po_v.at[pp], xp_hbm.at[pp * (B * C) + qb], osem)
        return 0

    lax.fori_loop(0, PAIRS_PER_W, pair_body, 0)
    drain_outs(0)


def _sc_body(x_hbm, q_hbm, roff_hbm, coff_hbm, w1_hbm, w2_hbm, out_hbm,
             q_v, roff_v, coff_v, w1_v, w2_v, tap_v, out_v, sem_a, sem_b,
             osem):
    cid = lax.axis_index("c")
    sid = lax.axis_index("s")
    wid = sid * 2 + cid

    pltpu.sync_copy(q_hbm, q_v)
    pltpu.sync_copy(roff_hbm, roff_v)
    pltpu.sync_copy(coff_hbm, coff_v)
    pltpu.sync_copy(w1_hbm, w1_v)
    pltpu.sync_copy(w2_hbm, w2_v)

    def pair_body(k, _):
        b = k & 3
        oc = wid * 3 + lax.shift_right_logical(k, 2)
        base = oc * 9

        qvec = q_v[pl.ds(base, 16)]
        rvec = roff_v[pl.ds(base, 16)]
        cvec = coff_v[pl.ds(base, 16)]
        w1vec = w1_v[pl.ds(base, 16)]
        w2vec = w2_v[pl.ds(oc * 3, 16)]
        planes = [qvec[t] + b * C for t in range(9)]
        roffs = [rvec[t] for t in range(9)]
        coffs = [cvec[t] for t in range(9)]
        w1s = [jnp.full((16,), w1vec[t], jnp.float32) for t in range(9)]
        w2s = [jnp.full((16,), w2vec[i], jnp.float32) for i in range(3)]

        def fetch(ch, slot, sem):
            h0 = ch * R
            return [pltpu.async_copy(
                x_hbm.at[planes[t], pl.ds(h0 + roffs[t], R), :],
                tap_v.at[slot, t], sem) for t in range(9)]

        def compute(slot, ch):
            def row_body(r, _):
                for cc in range(WO // 16):
                    acc = None
                    for i in range(3):
                        m = None
                        for j in range(3):
                            t = 3 * i + j
                            v = tap_v[slot, t, r, pl.ds(coffs[t] + cc * 16, 16)]
                            v = v - w1s[t]
                            m = v if m is None else jnp.maximum(m, v)
                        m = m - w2s[i]
                        acc = m if acc is None else jnp.minimum(acc, m)
                    out_v[slot, r, pl.ds(cc * 16, 16)] = acc
                return 0

            lax.fori_loop(0, R, row_body, 0)
            pltpu.async_copy(
                out_v.at[slot], out_hbm.at[b, oc, pl.ds(ch * R, R), :],
                osem).wait()

        cps = fetch(0, 0, sem_a)
        for ch2 in range(NCHUNK // 2):
            e = 2 * ch2
            cps_o = fetch(e + 1, 1, sem_b)
            for cp in cps:
                cp.wait()
            compute(0, e)
            cps = fetch(e + 2, 0, sem_a) if e + 2 < NCHUNK else []
            for cp in cps_o:
                cp.wait()
            compute(1, e + 1)
        return 0

    lax.fori_loop(0, PAIRS_PER_W, pair_body, 0)


def kernel(x, conn, w1, w2):
    mesh = plsc.VectorSubcoreMesh(core_axis_name="c", subcore_axis_name="s")

    deint = pl.kernel(
        _deint_body,
        out_type=jax.ShapeDtypeStruct((2 * 2 * B * C, HP, WPAD), jnp.float32),
        mesh=mesh,
        compiler_params=pltpu.CompilerParams(use_tc_tiling_on_sc=False, needs_layout_passes=False),
        scratch_types=[
            pltpu.VMEM((224, 224), jnp.float32),
            pltpu.VMEM((4, HP, WPAD), jnp.float32),
            pltpu.SemaphoreType.DMA,
            pltpu.SemaphoreType.DMA,
        ],
    )
    X = deint(x)

    conn = conn.astype(jnp.int32)
    c = conn // 9
    rem = conn - c * 9
    di = rem // 3
    dj = rem - di * 3
    qbase = ((di % 2) * 2 + dj % 2) * (B * C) + c
    pad16 = lambda a: jnp.pad(a, (0, 16))
    q_p = pad16(qbase)
    roff_p = pad16(di // 2)
    coff_p = pad16(dj // 2)
    w1_p = pad16(w1.reshape(-1))
    w2_p = pad16(w2.reshape(-1))

    run = pl.kernel(
        _sc_body,
        out_type=jax.ShapeDtypeStruct((B, OC, HO, WO), jnp.float32),
        mesh=mesh,
        compiler_params=pltpu.CompilerParams(use_tc_tiling_on_sc=False, needs_layout_passes=False),
        scratch_types=[
            pltpu.VMEM((OC * 9 + 16,), jnp.int32),
            pltpu.VMEM((OC * 9 + 16,), jnp.int32),
            pltpu.VMEM((OC * 9 + 16,), jnp.int32),
            pltpu.VMEM((OC * 9 + 16,), jnp.float32),
            pltpu.VMEM((OC * 3 + 16,), jnp.float32),
            pltpu.VMEM((2, 9, R, WPAD), jnp.float32),
            pltpu.VMEM((2, R, WO), jnp.float32),
            pltpu.SemaphoreType.DMA,
            pltpu.SemaphoreType.DMA,
            pltpu.SemaphoreType.DMA,
        ],
    )
    return run(X, q_p, roff_p, coff_p, w1_p, w2_p)


# speedup vs baseline: 13.4076x; 1.0309x over previous
import jax
import jax.numpy as jnp
from jax import lax
from jax.experimental import pallas as pl
from jax.experimental.pallas import tpu as pltpu, tpu_sc as plsc

B = 4
C = 96
OC = 96
HO = 112
WO = 112
HP = 113
WPAD = 128
R = 28
NCHUNK = HO // R
NWORK = 32
PAIRS_PER_W = (B * OC) // NWORK


def _deint_body(x_hbm, xp_hbm, src_a, src_b, po_v, isem_a, isem_b, osem):
    cid = lax.axis_index("c")
    sid = lax.axis_index("s")
    wid = sid * 2 + cid

    iota = lax.iota(jnp.int32, 16)
    i2 = iota + iota
    zero = jnp.zeros((16,), jnp.int32)
    c223 = jnp.full((16,), 223, jnp.int32)
    cidxs = [[jnp.minimum(jnp.maximum(
        i2 + jnp.full((16,), 32 * kv + pc - 1, jnp.int32), zero), c223)
        for kv in range(WPAD // 16)] for pc in range(2)]

    def drain_outs():
        for pp in range(4):
            pltpu.make_async_copy(
                po_v.at[pp], xp_hbm.at[pp * (B * C)], osem).wait()

    def issue_a(b, c):
        pltpu.async_copy(x_hbm.at[b, c, pl.ds(0, 113), :], src_a, isem_a)

    def issue_b(b, c):
        pltpu.async_copy(x_hbm.at[b, c, pl.ds(113, 111), :], src_b, isem_b)

    issue_a(0, wid * 3)
    issue_b(0, wid * 3)

    def pair_body(k, _):
        b = k & 3
        c = wid * 3 + lax.shift_right_logical(k, 2)
        kn = k + 1
        bn = kn & 3
        cn = wid * 3 + lax.shift_right_logical(kn, 2)

        pltpu.make_async_copy(
            x_hbm.at[0, 0, pl.ds(0, 113), :], src_a, isem_a).wait()

        @pl.when(k > 0)
        def _():
            drain_outs()

        def row_body_a(i, _):
            s0 = jnp.maximum(i + i - 1, 0)
            s1 = i + i
            srows = (jnp.full((16,), s0, jnp.int32),
                     jnp.full((16,), s1, jnp.int32))
            for pp in range(4):
                pr, pc = pp >> 1, pp & 1
                for kv in range(WPAD // 16):
                    po_v[pp, i, pl.ds(kv * 16, 16)] = plsc.load_gather(
                        src_a, [srows[pr], cidxs[pc][kv]])
            return 0

        lax.fori_loop(0, 57, row_body_a, 0)

        @pl.when(kn < PAIRS_PER_W)
        def _():
            issue_a(bn, cn)

        pltpu.make_async_copy(
            x_hbm.at[0, 0, pl.ds(113, 111), :], src_b, isem_b).wait()

        def row_body_b(i, _):
            s0 = i + i - 1 - 113
            s1 = jnp.minimum(i + i, 223) - 113
            srows = (jnp.full((16,), s0, jnp.int32),
                     jnp.full((16,), s1, jnp.int32))
            for pp in range(4):
                pr, pc = pp >> 1, pp & 1
                for kv in range(WPAD // 16):
                    po_v[pp, i, pl.ds(kv * 16, 16)] = plsc.load_gather(
                        src_b, [srows[pr], cidxs[pc][kv]])
            return 0

        lax.fori_loop(57, HP, row_body_b, 0)

        qb = b * C + c
        for pp in range(4):
            pltpu.async_copy(po_v.at[pp], xp_hbm.at[pp * (B * C) + qb], osem)

        @pl.when(kn < PAIRS_PER_W)
        def _():
            issue_b(bn, cn)
        return 0

    lax.fori_loop(0, PAIRS_PER_W, pair_body, 0)
    drain_outs()


def _sc_body(x_hbm, q_hbm, roff_hbm, coff_hbm, w1_hbm, w2_hbm, out_hbm,
             q_v, roff_v, coff_v, w1_v, w2_v, tap_v, out_v, sem_a, sem_b,
             osem):
    cid = lax.axis_index("c")
    sid = lax.axis_index("s")
    wid = sid * 2 + cid

    pltpu.sync_copy(q_hbm, q_v)
    pltpu.sync_copy(roff_hbm, roff_v)
    pltpu.sync_copy(coff_hbm, coff_v)
    pltpu.sync_copy(w1_hbm, w1_v)
    pltpu.sync_copy(w2_hbm, w2_v)

    def pair_body(k, _):
        b = k & 3
        oc = wid * 3 + lax.shift_right_logical(k, 2)
        base = oc * 9

        qvec = q_v[pl.ds(base, 16)]
        rvec = roff_v[pl.ds(base, 16)]
        cvec = coff_v[pl.ds(base, 16)]
        w1vec = w1_v[pl.ds(base, 16)]
        w2vec = w2_v[pl.ds(oc * 3, 16)]
        planes = [qvec[t] + b * C for t in range(9)]
        roffs = [rvec[t] for t in range(9)]
        coffs = [cvec[t] for t in range(9)]
        w1s = [jnp.full((16,), w1vec[t], jnp.float32) for t in range(9)]
        w2s = [jnp.full((16,), w2vec[i], jnp.float32) for i in range(3)]

        def fetch(ch, slot, sem):
            h0 = ch * R
            return [pltpu.async_copy(
                x_hbm.at[planes[t], pl.ds(h0 + roffs[t], R), :],
                tap_v.at[slot, t], sem) for t in range(9)]

        def compute(slot, ch):
            def row_body(r, _):
                for cc in range(WO // 16):
                    acc = None
                    for i in range(3):
                        m = None
                        for j in range(3):
                            t = 3 * i + j
                            v = tap_v[slot, t, r, pl.ds(coffs[t] + cc * 16, 16)]
                            v = v - w1s[t]
                            m = v if m is None else jnp.maximum(m, v)
                        m = m - w2s[i]
                        acc = m if acc is None else jnp.minimum(acc, m)
                    out_v[slot, r, pl.ds(cc * 16, 16)] = acc
                return 0

            lax.fori_loop(0, R, row_body, 0)
            pltpu.async_copy(
                out_v.at[slot], out_hbm.at[b, oc, pl.ds(ch * R, R), :],
                osem).wait()

        cps = fetch(0, 0, sem_a)
        for ch2 in range(NCHUNK // 2):
            e = 2 * ch2
            cps_o = fetch(e + 1, 1, sem_b)
            for cp in cps:
                cp.wait()
            compute(0, e)
            cps = fetch(e + 2, 0, sem_a) if e + 2 < NCHUNK else []
            for cp in cps_o:
                cp.wait()
            compute(1, e + 1)
        return 0

    lax.fori_loop(0, PAIRS_PER_W, pair_body, 0)


def kernel(x, conn, w1, w2):
    mesh = plsc.VectorSubcoreMesh(core_axis_name="c", subcore_axis_name="s")

    deint = pl.kernel(
        _deint_body,
        out_type=jax.ShapeDtypeStruct((2 * 2 * B * C, HP, WPAD), jnp.float32),
        mesh=mesh,
        compiler_params=pltpu.CompilerParams(use_tc_tiling_on_sc=False, needs_layout_passes=False),
        scratch_types=[
            pltpu.VMEM((113, 224), jnp.float32),
            pltpu.VMEM((111, 224), jnp.float32),
            pltpu.VMEM((4, HP, WPAD), jnp.float32),
            pltpu.SemaphoreType.DMA,
            pltpu.SemaphoreType.DMA,
            pltpu.SemaphoreType.DMA,
        ],
    )
    X = deint(x)

    conn = conn.astype(jnp.int32)
    c = conn // 9
    rem = conn - c * 9
    di = rem // 3
    dj = rem - di * 3
    qbase = ((di % 2) * 2 + dj % 2) * (B * C) + c
    pad16 = lambda a: jnp.pad(a, (0, 16))
    q_p = pad16(qbase)
    roff_p = pad16(di // 2)
    coff_p = pad16(dj // 2)
    w1_p = pad16(w1.reshape(-1))
    w2_p = pad16(w2.reshape(-1))

    run = pl.kernel(
        _sc_body,
        out_type=jax.ShapeDtypeStruct((B, OC, HO, WO), jnp.float32),
        mesh=mesh,
        compiler_params=pltpu.CompilerParams(use_tc_tiling_on_sc=False, needs_layout_passes=False),
        scratch_types=[
            pltpu.VMEM((OC * 9 + 16,), jnp.int32),
            pltpu.VMEM((OC * 9 + 16,), jnp.int32),
            pltpu.VMEM((OC * 9 + 16,), jnp.int32),
            pltpu.VMEM((OC * 9 + 16,), jnp.float32),
            pltpu.VMEM((OC * 3 + 16,), jnp.float32),
            pltpu.VMEM((2, 9, R, WPAD), jnp.float32),
            pltpu.VMEM((2, R, WO), jnp.float32),
            pltpu.SemaphoreType.DMA,
            pltpu.SemaphoreType.DMA,
            pltpu.SemaphoreType.DMA,
        ],
    )
    return run(X, q_p, roff_p, coff_p, w1_p, w2_p)


# speedup vs baseline: 13.7181x; 1.0232x over previous
import jax
import jax.numpy as jnp
from jax import lax
from jax.experimental import pallas as pl
from jax.experimental.pallas import tpu as pltpu, tpu_sc as plsc

B = 4
C = 96
OC = 96
HO = 112
WO = 112
HP = 113
WPAD = 128
R = 28
NCHUNK = HO // R
NWORK = 32
PAIRS_PER_W = (B * OC) // NWORK


def _deint_body(x_hbm, xp_hbm, src_a, src_b, po_v, isem_a, isem_b, osem):
    cid = lax.axis_index("c")
    sid = lax.axis_index("s")
    wid = sid * 2 + cid

    iota = lax.iota(jnp.int32, 16)
    zero = jnp.zeros((16,), jnp.int32)
    one = jnp.full((16,), 1, jnp.int32)
    mask_even = (iota & one) == zero
    mask_odd = jnp.logical_not(mask_even)
    r112 = jnp.full((16,), 112, jnp.int32)

    def drain_outs():
        for pp in range(4):
            pltpu.make_async_copy(
                po_v.at[pp], xp_hbm.at[pp * (B * C)], osem).wait()

    def issue_a(b, c):
        pltpu.async_copy(x_hbm.at[b, c, pl.ds(0, 113), :], src_a, isem_a)

    def issue_b(b, c):
        pltpu.async_copy(x_hbm.at[b, c, pl.ds(113, 111), :], src_b, isem_b)

    issue_a(0, wid * 3)
    issue_b(0, wid * 3)

    def pair_body(k, _):
        b = k & 3
        c = wid * 3 + lax.shift_right_logical(k, 2)
        kn = k + 1
        bn = kn & 3
        cn = wid * 3 + lax.shift_right_logical(kn, 2)

        pltpu.make_async_copy(
            x_hbm.at[0, 0, pl.ds(0, 113), :], src_a, isem_a).wait()

        @pl.when(k > 0)
        def _():
            drain_outs()

        def make_row_body(src, soff):
            def row_body(ls, _):
                s = ls + soff
                trow = lax.shift_right_logical(s + 1, 1)
                pr = 1 - (s & 1)
                ppe = pr + pr + 1
                ppo = pr + pr
                for m in range(14):
                    v = src[ls, pl.ds(16 * m, 16)]
                    plsc.store_compressed(
                        po_v.at[ppe, trow, pl.ds(8 * m, 16)], v, mask=mask_even)
                    plsc.store_compressed(
                        po_v.at[ppo, trow, pl.ds(8 * m + 1, 16)], v, mask=mask_odd)
                return 0
            return row_body

        lax.fori_loop(0, 113, make_row_body(src_a, 0), 0)

        @pl.when(kn < PAIRS_PER_W)
        def _():
            issue_a(bn, cn)

        pltpu.make_async_copy(
            x_hbm.at[0, 0, pl.ds(113, 111), :], src_b, isem_b).wait()

        lax.fori_loop(0, 111, make_row_body(src_b, 113), 0)

        for pc in range(2):
            for kv in range(WPAD // 16):
                po_v[pc, 0, pl.ds(kv * 16, 16)] = \
                    po_v[2 + pc, 0, pl.ds(kv * 16, 16)]
                po_v[2 + pc, 112, pl.ds(kv * 16, 16)] = \
                    po_v[pc, 112, pl.ds(kv * 16, 16)]
        for pr in range(2):
            for kv in range(8):
                rows = jnp.minimum(jnp.full((16,), 16 * kv, jnp.int32) + iota,
                                   r112)
                ppa = jnp.full((16,), pr + pr, jnp.int32)
                ppb = jnp.full((16,), pr + pr + 1, jnp.int32)
                col0 = zero
                col112 = r112
                v0 = plsc.load_gather(po_v, [ppb, rows, col0])
                plsc.store_scatter(po_v, [ppa, rows, col0], v0)
                v1 = plsc.load_gather(po_v, [ppa, rows, col112])
                plsc.store_scatter(po_v, [ppb, rows, col112], v1)

        qb = b * C + c
        for pp in range(4):
            pltpu.async_copy(po_v.at[pp], xp_hbm.at[pp * (B * C) + qb], osem)

        @pl.when(kn < PAIRS_PER_W)
        def _():
            issue_b(bn, cn)
        return 0

    lax.fori_loop(0, PAIRS_PER_W, pair_body, 0)
    drain_outs()


def _sc_body(x_hbm, q_hbm, roff_hbm, coff_hbm, w1_hbm, w2_hbm, out_hbm,
             q_v, roff_v, coff_v, w1_v, w2_v, tap_v, out_v, sem_a, sem_b,
             osem):
    cid = lax.axis_index("c")
    sid = lax.axis_index("s")
    wid = sid * 2 + cid

    pltpu.sync_copy(q_hbm, q_v)
    pltpu.sync_copy(roff_hbm, roff_v)
    pltpu.sync_copy(coff_hbm, coff_v)
    pltpu.sync_copy(w1_hbm, w1_v)
    pltpu.sync_copy(w2_hbm, w2_v)

    def pair_body(k, _):
        b = k & 3
        oc = wid * 3 + lax.shift_right_logical(k, 2)
        base = oc * 9

        qvec = q_v[pl.ds(base, 16)]
        rvec = roff_v[pl.ds(base, 16)]
        cvec = coff_v[pl.ds(base, 16)]
        w1vec = w1_v[pl.ds(base, 16)]
        w2vec = w2_v[pl.ds(oc * 3, 16)]
        planes = [qvec[t] + b * C for t in range(9)]
        roffs = [rvec[t] for t in range(9)]
        coffs = [cvec[t] for t in range(9)]
        w1s = [jnp.full((16,), w1vec[t], jnp.float32) for t in range(9)]
        w2s = [jnp.full((16,), w2vec[i], jnp.float32) for i in range(3)]

        def fetch(ch, slot, sem):
            h0 = ch * R
            return [pltpu.async_copy(
                x_hbm.at[planes[t], pl.ds(h0 + roffs[t], R), :],
                tap_v.at[slot, t], sem) for t in range(9)]

        def compute(slot, ch):
            def row_body(r, _):
                for cc in range(WO // 16):
                    acc = None
                    for i in range(3):
                        m = None
                        for j in range(3):
                            t = 3 * i + j
                            v = tap_v[slot, t, r, pl.ds(coffs[t] + cc * 16, 16)]
                            v = v - w1s[t]
                            m = v if m is None else jnp.maximum(m, v)
                        m = m - w2s[i]
                        acc = m if acc is None else jnp.minimum(acc, m)
                    out_v[slot, r, pl.ds(cc * 16, 16)] = acc
                return 0

            lax.fori_loop(0, R, row_body, 0)
            pltpu.async_copy(
                out_v.at[slot], out_hbm.at[b, oc, pl.ds(ch * R, R), :],
                osem).wait()

        cps = fetch(0, 0, sem_a)
        for ch2 in range(NCHUNK // 2):
            e = 2 * ch2
            cps_o = fetch(e + 1, 1, sem_b)
            for cp in cps:
                cp.wait()
            compute(0, e)
            cps = fetch(e + 2, 0, sem_a) if e + 2 < NCHUNK else []
            for cp in cps_o:
                cp.wait()
            compute(1, e + 1)
        return 0

    lax.fori_loop(0, PAIRS_PER_W, pair_body, 0)


def kernel(x, conn, w1, w2):
    mesh = plsc.VectorSubcoreMesh(core_axis_name="c", subcore_axis_name="s")

    deint = pl.kernel(
        _deint_body,
        out_type=jax.ShapeDtypeStruct((2 * 2 * B * C, HP, WPAD), jnp.float32),
        mesh=mesh,
        compiler_params=pltpu.CompilerParams(use_tc_tiling_on_sc=False, needs_layout_passes=False),
        scratch_types=[
            pltpu.VMEM((113, 224), jnp.float32),
            pltpu.VMEM((111, 224), jnp.float32),
            pltpu.VMEM((4, HP, WPAD), jnp.float32),
            pltpu.SemaphoreType.DMA,
            pltpu.SemaphoreType.DMA,
            pltpu.SemaphoreType.DMA,
        ],
    )
    X = deint(x)

    conn = conn.astype(jnp.int32)
    c = conn // 9
    rem = conn - c * 9
    di = rem // 3
    dj = rem - di * 3
    qbase = ((di % 2) * 2 + dj % 2) * (B * C) + c
    pad16 = lambda a: jnp.pad(a, (0, 16))
    q_p = pad16(qbase)
    roff_p = pad16(di // 2)
    coff_p = pad16(dj // 2)
    w1_p = pad16(w1.reshape(-1))
    w2_p = pad16(w2.reshape(-1))

    run = pl.kernel(
        _sc_body,
        out_type=jax.ShapeDtypeStruct((B, OC, HO, WO), jnp.float32),
        mesh=mesh,
        compiler_params=pltpu.CompilerParams(use_tc_tiling_on_sc=False, needs_layout_passes=False),
        scratch_types=[
            pltpu.VMEM((OC * 9 + 16,), jnp.int32),
            pltpu.VMEM((OC * 9 + 16,), jnp.int32),
            pltpu.VMEM((OC * 9 + 16,), jnp.int32),
            pltpu.VMEM((OC * 9 + 16,), jnp.float32),
            pltpu.VMEM((OC * 3 + 16,), jnp.float32),
            pltpu.VMEM((2, 9, R, WPAD), jnp.float32),
            pltpu.VMEM((2, R, WO), jnp.float32),
            pltpu.SemaphoreType.DMA,
            pltpu.SemaphoreType.DMA,
            pltpu.SemaphoreType.DMA,
        ],
    )
    return run(X, q_p, roff_p, coff_p, w1_p, w2_p)


# speedup vs baseline: 14.1555x; 1.0319x over previous
import jax
import jax.numpy as jnp
from jax import lax
from jax.experimental import pallas as pl
from jax.experimental.pallas import tpu as pltpu, tpu_sc as plsc

B = 4
C = 96
OC = 96
HO = 112
WO = 112
HP = 113
WPAD = 128
R = 28
NCHUNK = HO // R
NWORK = 32
PAIRS_PER_W = (B * OC) // NWORK


def _deint_body(x_hbm, xp_hbm, src_a, src_b, po_v, isem_a, isem_b, osem):
    cid = lax.axis_index("c")
    sid = lax.axis_index("s")
    wid = sid * 2 + cid

    iota = lax.iota(jnp.int32, 16)
    zero = jnp.zeros((16,), jnp.int32)
    one = jnp.full((16,), 1, jnp.int32)
    mask_even = (iota & one) == zero
    mask_odd = jnp.logical_not(mask_even)
    r112 = jnp.full((16,), 112, jnp.int32)

    def drain_outs():
        for pp in range(4):
            pltpu.make_async_copy(
                po_v.at[pp, pl.ds(0, 57)],
                xp_hbm.at[pp * (B * C), pl.ds(0, 57)], osem).wait()
            pltpu.make_async_copy(
                po_v.at[pp, pl.ds(57, 56)],
                xp_hbm.at[pp * (B * C), pl.ds(57, 56)], osem).wait()

    def col_fix(base_rows):
        for pr in range(2):
            for base in base_rows:
                rows = jnp.minimum(
                    jnp.full((16,), base, jnp.int32) + iota, r112)
                ppa = jnp.full((16,), pr + pr, jnp.int32)
                ppb = jnp.full((16,), pr + pr + 1, jnp.int32)
                v0 = plsc.load_gather(po_v, [ppb, rows, zero])
                plsc.store_scatter(po_v, [ppa, rows, zero], v0)
                v1 = plsc.load_gather(po_v, [ppa, rows, r112])
                plsc.store_scatter(po_v, [ppb, rows, r112], v1)

    def issue_a(b, c):
        pltpu.async_copy(x_hbm.at[b, c, pl.ds(0, 113), :], src_a, isem_a)

    def issue_b(b, c):
        pltpu.async_copy(x_hbm.at[b, c, pl.ds(113, 111), :], src_b, isem_b)

    issue_a(0, wid * 3)
    issue_b(0, wid * 3)

    def pair_body(k, _):
        b = k & 3
        c = wid * 3 + lax.shift_right_logical(k, 2)
        kn = k + 1
        bn = kn & 3
        cn = wid * 3 + lax.shift_right_logical(kn, 2)

        pltpu.make_async_copy(
            x_hbm.at[0, 0, pl.ds(0, 113), :], src_a, isem_a).wait()

        @pl.when(k > 0)
        def _():
            drain_outs()

        def make_row_body(src, soff):
            def row_body(ls, _):
                s = ls + soff
                trow = lax.shift_right_logical(s + 1, 1)
                pr = 1 - (s & 1)
                ppe = pr + pr + 1
                ppo = pr + pr
                for m in range(14):
                    v = src[ls, pl.ds(16 * m, 16)]
                    plsc.store_compressed(
                        po_v.at[ppe, trow, pl.ds(8 * m, 16)], v, mask=mask_even)
                    plsc.store_compressed(
                        po_v.at[ppo, trow, pl.ds(8 * m + 1, 16)], v, mask=mask_odd)
                return 0
            return row_body

        lax.fori_loop(0, 113, make_row_body(src_a, 0), 0)

        @pl.when(kn < PAIRS_PER_W)
        def _():
            issue_a(bn, cn)

        qb = b * C + c
        for pc in range(2):
            for kv in range(WPAD // 16):
                po_v[pc, 0, pl.ds(kv * 16, 16)] = \
                    po_v[2 + pc, 0, pl.ds(kv * 16, 16)]
        col_fix((0, 16, 32, 48))
        for pp in range(4):
            pltpu.async_copy(po_v.at[pp, pl.ds(0, 57)],
                             xp_hbm.at[pp * (B * C) + qb, pl.ds(0, 57)], osem)

        pltpu.make_async_copy(
            x_hbm.at[0, 0, pl.ds(113, 111), :], src_b, isem_b).wait()

        lax.fori_loop(0, 111, make_row_body(src_b, 113), 0)

        for pc in range(2):
            for kv in range(WPAD // 16):
                po_v[2 + pc, 112, pl.ds(kv * 16, 16)] = \
                    po_v[pc, 112, pl.ds(kv * 16, 16)]
        col_fix((57, 73, 89, 105))
        for pp in range(4):
            pltpu.async_copy(po_v.at[pp, pl.ds(57, 56)],
                             xp_hbm.at[pp * (B * C) + qb, pl.ds(57, 56)], osem)

        @pl.when(kn < PAIRS_PER_W)
        def _():
            issue_b(bn, cn)
        return 0

    lax.fori_loop(0, PAIRS_PER_W, pair_body, 0)
    drain_outs()


def _sc_body(x_hbm, q_hbm, roff_hbm, coff_hbm, w1_hbm, w2_hbm, out_hbm,
             q_v, roff_v, coff_v, w1_v, w2_v, tap_v, out_v, sem_a, sem_b,
             osem):
    cid = lax.axis_index("c")
    sid = lax.axis_index("s")
    wid = sid * 2 + cid

    pltpu.sync_copy(q_hbm, q_v)
    pltpu.sync_copy(roff_hbm, roff_v)
    pltpu.sync_copy(coff_hbm, coff_v)
    pltpu.sync_copy(w1_hbm, w1_v)
    pltpu.sync_copy(w2_hbm, w2_v)

    def pair_body(k, _):
        b = k & 3
        oc = wid * 3 + lax.shift_right_logical(k, 2)
        base = oc * 9

        qvec = q_v[pl.ds(base, 16)]
        rvec = roff_v[pl.ds(base, 16)]
        cvec = coff_v[pl.ds(base, 16)]
        w1vec = w1_v[pl.ds(base, 16)]
        w2vec = w2_v[pl.ds(oc * 3, 16)]
        planes = [qvec[t] + b * C for t in range(9)]
        roffs = [rvec[t] for t in range(9)]
        coffs = [cvec[t] for t in range(9)]
        w1s = [jnp.full((16,), w1vec[t], jnp.float32) for t in range(9)]
        w2s = [jnp.full((16,), w2vec[i], jnp.float32) for i in range(3)]

        def fetch(ch, slot, sem):
            h0 = ch * R
            return [pltpu.async_copy(
                x_hbm.at[planes[t], pl.ds(h0 + roffs[t], R), :],
                tap_v.at[slot, t], sem) for t in range(9)]

        def compute(slot, ch):
            def row_body(r, _):
                for cc in range(WO // 16):
                    acc = None
                    for i in range(3):
                        m = None
                        for j in range(3):
                            t = 3 * i + j
                            v = tap_v[slot, t, r, pl.ds(coffs[t] + cc * 16, 16)]
                            v = v - w1s[t]
                            m = v if m is None else jnp.maximum(m, v)
                        m = m - w2s[i]
                        acc = m if acc is None else jnp.minimum(acc, m)
                    out_v[slot, r, pl.ds(cc * 16, 16)] = acc
                return 0

            lax.fori_loop(0, R, row_body, 0)
            pltpu.async_copy(
                out_v.at[slot], out_hbm.at[b, oc, pl.ds(ch * R, R), :],
                osem).wait()

        cps = fetch(0, 0, sem_a)
        for ch2 in range(NCHUNK // 2):
            e = 2 * ch2
            cps_o = fetch(e + 1, 1, sem_b)
            for cp in cps:
                cp.wait()
            compute(0, e)
            cps = fetch(e + 2, 0, sem_a) if e + 2 < NCHUNK else []
            for cp in cps_o:
                cp.wait()
            compute(1, e + 1)
        return 0

    lax.fori_loop(0, PAIRS_PER_W, pair_body, 0)


def kernel(x, conn, w1, w2):
    mesh = plsc.VectorSubcoreMesh(core_axis_name="c", subcore_axis_name="s")

    deint = pl.kernel(
        _deint_body,
        out_type=jax.ShapeDtypeStruct((2 * 2 * B * C, HP, WPAD), jnp.float32),
        mesh=mesh,
        compiler_params=pltpu.CompilerParams(use_tc_tiling_on_sc=False, needs_layout_passes=False),
        scratch_types=[
            pltpu.VMEM((113, 224), jnp.float32),
            pltpu.VMEM((111, 224), jnp.float32),
            pltpu.VMEM((4, HP, WPAD), jnp.float32),
            pltpu.SemaphoreType.DMA,
            pltpu.SemaphoreType.DMA,
            pltpu.SemaphoreType.DMA,
        ],
    )
    X = deint(x)

    conn = conn.astype(jnp.int32)
    c = conn // 9
    rem = conn - c * 9
    di = rem // 3
    dj = rem - di * 3
    qbase = ((di % 2) * 2 + dj % 2) * (B * C) + c
    pad16 = lambda a: jnp.pad(a, (0, 16))
    q_p = pad16(qbase)
    roff_p = pad16(di // 2)
    coff_p = pad16(dj // 2)
    w1_p = pad16(w1.reshape(-1))
    w2_p = pad16(w2.reshape(-1))

    run = pl.kernel(
        _sc_body,
        out_type=jax.ShapeDtypeStruct((B, OC, HO, WO), jnp.float32),
        mesh=mesh,
        compiler_params=pltpu.CompilerParams(use_tc_tiling_on_sc=False, needs_layout_passes=False),
        scratch_types=[
            pltpu.VMEM((OC * 9 + 16,), jnp.int32),
            pltpu.VMEM((OC * 9 + 16,), jnp.int32),
            pltpu.VMEM((OC * 9 + 16,), jnp.int32),
            pltpu.VMEM((OC * 9 + 16,), jnp.float32),
            pltpu.VMEM((OC * 3 + 16,), jnp.float32),
            pltpu.VMEM((2, 9, R, WPAD), jnp.float32),
            pltpu.VMEM((2, R, WO), jnp.float32),
            pltpu.SemaphoreType.DMA,
            pltpu.SemaphoreType.DMA,
            pltpu.SemaphoreType.DMA,
        ],
    )
    return run(X, q_p, roff_p, coff_p, w1_p, w2_p)


# speedup vs baseline: 14.2076x; 1.0037x over previous
import jax
import jax.numpy as jnp
from jax import lax
from jax.experimental import pallas as pl
from jax.experimental.pallas import tpu as pltpu, tpu_sc as plsc

B = 4
C = 96
OC = 96
HO = 112
WO = 112
HP = 113
WPAD = 128
R = 28
NCHUNK = HO // R
NWORK = 32
PAIRS_PER_W = (B * OC) // NWORK


def _deint_body(x_hbm, xp_hbm, src_a, src_b, po_v, isem_a, isem_b, osem):
    cid = lax.axis_index("c")
    sid = lax.axis_index("s")
    wid = sid * 2 + cid

    iota = lax.iota(jnp.int32, 16)
    zero = jnp.zeros((16,), jnp.int32)
    one = jnp.full((16,), 1, jnp.int32)
    mask_even = (iota & one) == zero
    mask_odd = jnp.logical_not(mask_even)
    r112 = jnp.full((16,), 112, jnp.int32)

    def drain_outs():
        for pp in range(4):
            pltpu.make_async_copy(
                po_v.at[pp, pl.ds(0, 57)],
                xp_hbm.at[pp * (B * C), pl.ds(0, 57)], osem).wait()
            pltpu.make_async_copy(
                po_v.at[pp, pl.ds(57, 56)],
                xp_hbm.at[pp * (B * C), pl.ds(57, 56)], osem).wait()

    def col_fix(base_rows):
        for pr in range(2):
            for base in base_rows:
                rows = jnp.minimum(
                    jnp.full((16,), base, jnp.int32) + iota, r112)
                ppa = jnp.full((16,), pr + pr, jnp.int32)
                ppb = jnp.full((16,), pr + pr + 1, jnp.int32)
                v0 = plsc.load_gather(po_v, [ppb, rows, zero])
                plsc.store_scatter(po_v, [ppa, rows, zero], v0)
                v1 = plsc.load_gather(po_v, [ppa, rows, r112])
                plsc.store_scatter(po_v, [ppb, rows, r112], v1)

    def issue_a(b, c):
        pltpu.async_copy(x_hbm.at[b, c, pl.ds(0, 113), :], src_a, isem_a)

    def issue_b(b, c):
        pltpu.async_copy(x_hbm.at[b, c, pl.ds(113, 111), :], src_b, isem_b)

    issue_a(0, wid * 3)
    issue_b(0, wid * 3)

    def pair_body(k, _):
        b = k & 3
        c = wid * 3 + lax.shift_right_logical(k, 2)
        kn = k + 1
        bn = kn & 3
        cn = wid * 3 + lax.shift_right_logical(kn, 2)

        pltpu.make_async_copy(
            x_hbm.at[0, 0, pl.ds(0, 113), :], src_a, isem_a).wait()

        @pl.when(k > 0)
        def _():
            drain_outs()

        def emit_row(src, ls, ppe, ppo, trow):
            for m in range(14):
                v = src[ls, pl.ds(16 * m, 16)]
                plsc.store_compressed(
                    po_v.at[ppe, trow, pl.ds(8 * m, 16)], v, mask=mask_even)
                plsc.store_compressed(
                    po_v.at[ppo, trow, pl.ds(8 * m + 1, 16)], v, mask=mask_odd)

        def body_a(u, _):
            ls = u + u
            emit_row(src_a, ls, 3, 2, u)
            emit_row(src_a, ls + 1, 1, 0, u + 1)
            return 0

        lax.fori_loop(0, 56, body_a, 0)
        emit_row(src_a, 112, 3, 2, 56)

        @pl.when(kn < PAIRS_PER_W)
        def _():
            issue_a(bn, cn)

        qb = b * C + c
        for pc in range(2):
            for kv in range(WPAD // 16):
                po_v[pc, 0, pl.ds(kv * 16, 16)] = \
                    po_v[2 + pc, 0, pl.ds(kv * 16, 16)]
        col_fix((0, 16, 32, 48))
        for pp in range(4):
            pltpu.async_copy(po_v.at[pp, pl.ds(0, 57)],
                             xp_hbm.at[pp * (B * C) + qb, pl.ds(0, 57)], osem)

        pltpu.make_async_copy(
            x_hbm.at[0, 0, pl.ds(113, 111), :], src_b, isem_b).wait()

        emit_row(src_b, 0, 1, 0, 57)

        def body_b(u, _):
            ls = u + u + 1
            emit_row(src_b, ls, 3, 2, u + 57)
            emit_row(src_b, ls + 1, 1, 0, u + 58)
            return 0

        lax.fori_loop(0, 55, body_b, 0)

        for pc in range(2):
            for kv in range(WPAD // 16):
                po_v[2 + pc, 112, pl.ds(kv * 16, 16)] = \
                    po_v[pc, 112, pl.ds(kv * 16, 16)]
        col_fix((57, 73, 89, 105))
        for pp in range(4):
            pltpu.async_copy(po_v.at[pp, pl.ds(57, 56)],
                             xp_hbm.at[pp * (B * C) + qb, pl.ds(57, 56)], osem)

        @pl.when(kn < PAIRS_PER_W)
        def _():
            issue_b(bn, cn)
        return 0

    lax.fori_loop(0, PAIRS_PER_W, pair_body, 0)
    drain_outs()


def _sc_body(x_hbm, q_hbm, roff_hbm, coff_hbm, w1_hbm, w2_hbm, out_hbm,
             q_v, roff_v, coff_v, w1_v, w2_v, tap_v, out_v, sem_a, sem_b,
             osem):
    cid = lax.axis_index("c")
    sid = lax.axis_index("s")
    wid = sid * 2 + cid

    pltpu.sync_copy(q_hbm, q_v)
    pltpu.sync_copy(roff_hbm, roff_v)
    pltpu.sync_copy(coff_hbm, coff_v)
    pltpu.sync_copy(w1_hbm, w1_v)
    pltpu.sync_copy(w2_hbm, w2_v)

    def pair_body(k, _):
        b = k & 3
        oc = wid * 3 + lax.shift_right_logical(k, 2)
        base = oc * 9

        qvec = q_v[pl.ds(base, 16)]
        rvec = roff_v[pl.ds(base, 16)]
        cvec = coff_v[pl.ds(base, 16)]
        w1vec = w1_v[pl.ds(base, 16)]
        w2vec = w2_v[pl.ds(oc * 3, 16)]
        planes = [qvec[t] + b * C for t in range(9)]
        roffs = [rvec[t] for t in range(9)]
        coffs = [cvec[t] for t in range(9)]
        w1s = [jnp.full((16,), w1vec[t], jnp.float32) for t in range(9)]
        w2s = [jnp.full((16,), w2vec[i], jnp.float32) for i in range(3)]

        def fetch(ch, slot, sem):
            h0 = ch * R
            return [pltpu.async_copy(
                x_hbm.at[planes[t], pl.ds(h0 + roffs[t], R), :],
                tap_v.at[slot, t], sem) for t in range(9)]

        def compute(slot, ch):
            def row_body(r, _):
                for cc in range(WO // 16):
                    acc = None
                    for i in range(3):
                        m = None
                        for j in range(3):
                            t = 3 * i + j
                            v = tap_v[slot, t, r, pl.ds(coffs[t] + cc * 16, 16)]
                            v = v - w1s[t]
                            m = v if m is None else jnp.maximum(m, v)
                        m = m - w2s[i]
                        acc = m if acc is None else jnp.minimum(acc, m)
                    out_v[slot, r, pl.ds(cc * 16, 16)] = acc
                return 0

            lax.fori_loop(0, R, row_body, 0)
            pltpu.async_copy(
                out_v.at[slot], out_hbm.at[b, oc, pl.ds(ch * R, R), :],
                osem).wait()

        cps = fetch(0, 0, sem_a)
        for ch2 in range(NCHUNK // 2):
            e = 2 * ch2
            cps_o = fetch(e + 1, 1, sem_b)
            for cp in cps:
                cp.wait()
            compute(0, e)
            cps = fetch(e + 2, 0, sem_a) if e + 2 < NCHUNK else []
            for cp in cps_o:
                cp.wait()
            compute(1, e + 1)
        return 0

    lax.fori_loop(0, PAIRS_PER_W, pair_body, 0)


def kernel(x, conn, w1, w2):
    mesh = plsc.VectorSubcoreMesh(core_axis_name="c", subcore_axis_name="s")

    deint = pl.kernel(
        _deint_body,
        out_type=jax.ShapeDtypeStruct((2 * 2 * B * C, HP, WPAD), jnp.float32),
        mesh=mesh,
        compiler_params=pltpu.CompilerParams(use_tc_tiling_on_sc=False, needs_layout_passes=False),
        scratch_types=[
            pltpu.VMEM((113, 224), jnp.float32),
            pltpu.VMEM((111, 224), jnp.float32),
            pltpu.VMEM((4, HP, WPAD), jnp.float32),
            pltpu.SemaphoreType.DMA,
            pltpu.SemaphoreType.DMA,
            pltpu.SemaphoreType.DMA,
        ],
    )
    X = deint(x)

    conn = conn.astype(jnp.int32)
    c = conn // 9
    rem = conn - c * 9
    di = rem // 3
    dj = rem - di * 3
    qbase = ((di % 2) * 2 + dj % 2) * (B * C) + c
    pad16 = lambda a: jnp.pad(a, (0, 16))
    q_p = pad16(qbase)
    roff_p = pad16(di // 2)
    coff_p = pad16(dj // 2)
    w1_p = pad16(w1.reshape(-1))
    w2_p = pad16(w2.reshape(-1))

    run = pl.kernel(
        _sc_body,
        out_type=jax.ShapeDtypeStruct((B, OC, HO, WO), jnp.float32),
        mesh=mesh,
        compiler_params=pltpu.CompilerParams(use_tc_tiling_on_sc=False, needs_layout_passes=False),
        scratch_types=[
            pltpu.VMEM((OC * 9 + 16,), jnp.int32),
            pltpu.VMEM((OC * 9 + 16,), jnp.int32),
            pltpu.VMEM((OC * 9 + 16,), jnp.int32),
            pltpu.VMEM((OC * 9 + 16,), jnp.float32),
            pltpu.VMEM((OC * 3 + 16,), jnp.float32),
            pltpu.VMEM((2, 9, R, WPAD), jnp.float32),
            pltpu.VMEM((2, R, WO), jnp.float32),
            pltpu.SemaphoreType.DMA,
            pltpu.SemaphoreType.DMA,
            pltpu.SemaphoreType.DMA,
        ],
    )
    return run(X, q_p, roff_p, coff_p, w1_p, w2_p)
